# baseline (speedup 1.0000x reference)
import functools

import jax
import jax.numpy as jnp
from jax import lax
from jax.experimental import pallas as pl
from jax.experimental.pallas import tpu as pltpu
from jax.experimental.pallas import tpu_sc as plsc

BLOCK_B = 16
NBUF = 6
COL_CHUNKS = ((0, 25088), (25088, 25088), (50176, 25088), (75264, 24736))


def _outer_body(ti_ref, tl_ref, emb_ref, out_hbm, buf, sems):
    i = pl.program_id(0)
    n = pl.num_programs(0)
    slot = jax.lax.rem(i, NBUF)

    def copy(step, s, sub):
        c0, w = COL_CHUNKS[sub]
        return pltpu.make_async_copy(
            buf.at[s, :, pl.ds(c0, w)],
            out_hbm.at[pl.ds(step * BLOCK_B, BLOCK_B), pl.ds(c0, w)],
            sems.at[s, sub],
        )

    @pl.when(i >= NBUF)
    def _():
        for sub in range(len(COL_CHUNKS)):
            copy(i - NBUF, slot, sub).wait()

    rows = pl.ds(i * BLOCK_B, BLOCK_B)
    dts = ti_ref[rows, :] - tl_ref[rows, :]
    buf[slot] = dts * emb_ref[...]
    for sub in range(len(COL_CHUNKS)):
        copy(i, slot, sub).start()

    @pl.when(i == n - 1)
    def _():
        for k in range(NBUF):
            for sub in range(len(COL_CHUNKS)):
                copy(i, jax.lax.rem(i - k, NBUF), sub).wait()


def _outer_product(ti, tlast, emb_row):
    B = ti.shape[0]
    V = emb_row.shape[1]
    grid = B // BLOCK_B
    return pl.pallas_call(
        _outer_body,
        grid=(grid,),
        in_specs=[
            pl.BlockSpec((B, 1), lambda i: (0, 0)),
            pl.BlockSpec((B, 1), lambda i: (0, 0)),
            pl.BlockSpec((1, V), lambda i: (0, 0)),
        ],
        out_specs=pl.BlockSpec(memory_space=pl.ANY),
        out_shape=jax.ShapeDtypeStruct((B, V), jnp.float32),
        scratch_shapes=[
            pltpu.VMEM((NBUF, BLOCK_B, V), jnp.float32),
            pltpu.SemaphoreType.DMA((NBUF, len(COL_CHUNKS))),
        ],
    )(ti, tlast, emb_row)


BLOCK_V = 1000
NBUF_T = 8


def _outer_t_body(ti_ref, tl_ref, emb_ref, out_hbm, buf, sems):
    i = pl.program_id(0)
    n = pl.num_programs(0)
    slot = jax.lax.rem(i, NBUF_T)

    def copy(step, s):
        return pltpu.make_async_copy(
            buf.at[s],
            out_hbm.at[pl.ds(step * BLOCK_V, BLOCK_V), :],
            sems.at[s],
        )

    @pl.when(i >= NBUF_T)
    def _():
        copy(i - NBUF_T, slot).wait()

    dts = ti_ref[...] - tl_ref[...]
    buf[slot] = emb_ref[...] * dts
    copy(i, slot).start()

    @pl.when(i == n - 1)
    def _():
        for k in range(NBUF_T):
            copy(i, jax.lax.rem(i - k, NBUF_T)).wait()


def _outer_product_t(ti_row, tl_row, emb):
    V = emb.shape[0]
    B = ti_row.shape[1]
    grid = V // BLOCK_V
    return pl.pallas_call(
        _outer_t_body,
        grid=(grid,),
        in_specs=[
            pl.BlockSpec((1, B), lambda i: (0, 0)),
            pl.BlockSpec((1, B), lambda i: (0, 0)),
            pl.BlockSpec((BLOCK_V, 1), lambda i: (i, 0)),
        ],
        out_specs=pl.BlockSpec(memory_space=pl.ANY),
        out_shape=jax.ShapeDtypeStruct((V, B), jnp.float32),
        scratch_shapes=[
            pltpu.VMEM((NBUF_T, BLOCK_V, B), jnp.float32),
            pltpu.SemaphoreType.DMA((NBUF_T,)),
        ],
    )(ti_row, tl_row, emb)


@functools.lru_cache(maxsize=None)
def _make_sc_gather(B):
    info = plsc.get_sparse_core_info()
    NC, NS = info.num_cores, info.num_subcores
    NW = NC * NS
    b_per_w = B // NW
    mesh = plsc.VectorSubcoreMesh(core_axis_name="c", subcore_axis_name="s")

    @functools.partial(
        pl.kernel,
        mesh=mesh,
        out_type=jax.ShapeDtypeStruct((B,), jnp.float32),
        scratch_types=[
            pltpu.VMEM((b_per_w,), jnp.int32),
            pltpu.VMEM((b_per_w,), jnp.float32),
            pltpu.SemaphoreType.DMA,
        ],
    )
    def gather(idx_hbm, table_hbm, out_hbm, idx_v, rows_v, sem):
        wid = lax.axis_index("s") * NC + lax.axis_index("c")
        base = wid * b_per_w
        pltpu.sync_copy(idx_hbm.at[pl.ds(base, b_per_w)], idx_v)
        pltpu.async_copy(table_hbm.at[idx_v], rows_v, sem).wait()
        pltpu.sync_copy(rows_v, out_hbm.at[pl.ds(base, b_per_w)])

    return gather


def kernel(ti, tjs, ci, Cs, emb):
    B = ti.shape[0]
    V = emb.shape[0]
    tlast = tjs[:, -1:]
    emb_row = emb.reshape(1, V)
    mUT = _outer_product_t(ti.reshape(1, B), tlast.reshape(1, B), emb)
    mU = mUT.T
    mu_c = _make_sc_gather(B)(ci.reshape(B), emb.reshape(V))
    return mu_c.reshape(B, 1), mU


# speedup vs baseline: 1.0187x; 1.0187x over previous
import functools

import jax
import jax.numpy as jnp
from jax import lax
from jax.experimental import pallas as pl
from jax.experimental.pallas import tpu as pltpu
from jax.experimental.pallas import tpu_sc as plsc

BLOCK_B = 16
NBUF = 6
COL_CHUNKS = ((0, 25088), (25088, 25088), (50176, 25088), (75264, 24736))


def _outer_body(ti_ref, tl_ref, emb_ref, out_hbm, buf, sems):
    i = pl.program_id(0)
    n = pl.num_programs(0)
    slot = jax.lax.rem(i, NBUF)

    def copy(step, s, sub):
        c0, w = COL_CHUNKS[sub]
        return pltpu.make_async_copy(
            buf.at[s, :, pl.ds(c0, w)],
            out_hbm.at[pl.ds(step * BLOCK_B, BLOCK_B), pl.ds(c0, w)],
            sems.at[s, sub],
        )

    @pl.when(i >= NBUF)
    def _():
        for sub in range(len(COL_CHUNKS)):
            copy(i - NBUF, slot, sub).wait()

    rows = pl.ds(i * BLOCK_B, BLOCK_B)
    dts = ti_ref[rows, :] - tl_ref[rows, :]
    buf[slot] = dts * emb_ref[...]
    for sub in range(len(COL_CHUNKS)):
        copy(i, slot, sub).start()

    @pl.when(i == n - 1)
    def _():
        for k in range(NBUF):
            for sub in range(len(COL_CHUNKS)):
                copy(i, jax.lax.rem(i - k, NBUF), sub).wait()


def _outer_product(ti, tlast, emb_row):
    B = ti.shape[0]
    V = emb_row.shape[1]
    grid = B // BLOCK_B
    return pl.pallas_call(
        _outer_body,
        grid=(grid,),
        in_specs=[
            pl.BlockSpec((B, 1), lambda i: (0, 0)),
            pl.BlockSpec((B, 1), lambda i: (0, 0)),
            pl.BlockSpec((1, V), lambda i: (0, 0)),
        ],
        out_specs=pl.BlockSpec(memory_space=pl.ANY),
        out_shape=jax.ShapeDtypeStruct((B, V), jnp.float32),
        scratch_shapes=[
            pltpu.VMEM((NBUF, BLOCK_B, V), jnp.float32),
            pltpu.SemaphoreType.DMA((NBUF, len(COL_CHUNKS))),
        ],
    )(ti, tlast, emb_row)


BLOCK_V = 4096


def _outer_t_body(ti_ref, tl_ref, emb_ref, out_ref):
    dts = ti_ref[...] - tl_ref[...]
    out_ref[...] = emb_ref[...] * dts


def _outer_product_t(ti_row, tl_row, emb):
    V = emb.shape[0]
    B = ti_row.shape[1]
    grid = pl.cdiv(V, BLOCK_V)
    return pl.pallas_call(
        _outer_t_body,
        grid=(grid,),
        in_specs=[
            pl.BlockSpec((1, B), lambda i: (0, 0)),
            pl.BlockSpec((1, B), lambda i: (0, 0)),
            pl.BlockSpec((BLOCK_V, 1), lambda i: (i, 0)),
        ],
        out_specs=pl.BlockSpec((BLOCK_V, B), lambda i: (i, 0)),
        out_shape=jax.ShapeDtypeStruct((V, B), jnp.float32),
    )(ti_row, tl_row, emb)


@functools.lru_cache(maxsize=None)
def _make_sc_gather(B):
    info = plsc.get_sparse_core_info()
    NC, NS = info.num_cores, info.num_subcores
    NW = NC * NS
    b_per_w = B // NW
    mesh = plsc.VectorSubcoreMesh(core_axis_name="c", subcore_axis_name="s")

    @functools.partial(
        pl.kernel,
        mesh=mesh,
        out_type=jax.ShapeDtypeStruct((B,), jnp.float32),
        scratch_types=[
            pltpu.VMEM((b_per_w,), jnp.int32),
            pltpu.VMEM((b_per_w,), jnp.float32),
            pltpu.SemaphoreType.DMA,
        ],
    )
    def gather(idx_hbm, table_hbm, out_hbm, idx_v, rows_v, sem):
        wid = lax.axis_index("s") * NC + lax.axis_index("c")
        base = wid * b_per_w
        pltpu.sync_copy(idx_hbm.at[pl.ds(base, b_per_w)], idx_v)
        pltpu.async_copy(table_hbm.at[idx_v], rows_v, sem).wait()
        pltpu.sync_copy(rows_v, out_hbm.at[pl.ds(base, b_per_w)])

    return gather


def kernel(ti, tjs, ci, Cs, emb):
    B = ti.shape[0]
    V = emb.shape[0]
    tlast = tjs[:, -1:]
    emb_row = emb.reshape(1, V)
    mUT = _outer_product_t(ti.reshape(1, B), tlast.reshape(1, B), emb)
    mU = mUT.T
    mu_c = _make_sc_gather(B)(ci.reshape(B), emb.reshape(V))
    return mu_c.reshape(B, 1), mU


# speedup vs baseline: 1.0233x; 1.0045x over previous
import functools

import jax
import jax.numpy as jnp
from jax import lax
from jax.experimental import pallas as pl
from jax.experimental.pallas import tpu as pltpu
from jax.experimental.pallas import tpu_sc as plsc

BLOCK_B = 16
NBUF = 6
COL_CHUNKS = ((0, 25088), (25088, 25088), (50176, 25088), (75264, 24736))


def _outer_body(ti_ref, tl_ref, emb_ref, out_hbm, buf, sems):
    i = pl.program_id(0)
    n = pl.num_programs(0)
    slot = jax.lax.rem(i, NBUF)

    def copy(step, s, sub):
        c0, w = COL_CHUNKS[sub]
        return pltpu.make_async_copy(
            buf.at[s, :, pl.ds(c0, w)],
            out_hbm.at[pl.ds(step * BLOCK_B, BLOCK_B), pl.ds(c0, w)],
            sems.at[s, sub],
        )

    @pl.when(i >= NBUF)
    def _():
        for sub in range(len(COL_CHUNKS)):
            copy(i - NBUF, slot, sub).wait()

    rows = pl.ds(i * BLOCK_B, BLOCK_B)
    dts = ti_ref[rows, :] - tl_ref[rows, :]
    buf[slot] = dts * emb_ref[...]
    for sub in range(len(COL_CHUNKS)):
        copy(i, slot, sub).start()

    @pl.when(i == n - 1)
    def _():
        for k in range(NBUF):
            for sub in range(len(COL_CHUNKS)):
                copy(i, jax.lax.rem(i - k, NBUF), sub).wait()


def _outer_product(ti, tlast, emb_row):
    B = ti.shape[0]
    V = emb_row.shape[1]
    grid = B // BLOCK_B
    return pl.pallas_call(
        _outer_body,
        grid=(grid,),
        in_specs=[
            pl.BlockSpec((B, 1), lambda i: (0, 0)),
            pl.BlockSpec((B, 1), lambda i: (0, 0)),
            pl.BlockSpec((1, V), lambda i: (0, 0)),
        ],
        out_specs=pl.BlockSpec(memory_space=pl.ANY),
        out_shape=jax.ShapeDtypeStruct((B, V), jnp.float32),
        scratch_shapes=[
            pltpu.VMEM((NBUF, BLOCK_B, V), jnp.float32),
            pltpu.SemaphoreType.DMA((NBUF, len(COL_CHUNKS))),
        ],
    )(ti, tlast, emb_row)


BLOCK_V = 5000


def _outer_t_body(ti_ref, tl_ref, emb_ref, out_ref):
    dts = ti_ref[...] - tl_ref[...]
    out_ref[...] = emb_ref[...] * dts


def _outer_product_t(ti_row, tl_row, emb):
    V = emb.shape[0]
    B = ti_row.shape[1]
    grid = pl.cdiv(V, BLOCK_V)
    return pl.pallas_call(
        _outer_t_body,
        grid=(grid,),
        in_specs=[
            pl.BlockSpec((1, B), lambda i: (0, 0)),
            pl.BlockSpec((1, B), lambda i: (0, 0)),
            pl.BlockSpec((BLOCK_V, 1), lambda i: (i, 0)),
        ],
        out_specs=pl.BlockSpec((BLOCK_V, B), lambda i: (i, 0)),
        out_shape=jax.ShapeDtypeStruct((V, B), jnp.float32),
    )(ti_row, tl_row, emb)


@functools.lru_cache(maxsize=None)
def _make_sc_gather(B):
    info = plsc.get_sparse_core_info()
    NC, NS = info.num_cores, info.num_subcores
    NW = NC * NS
    b_per_w = B // NW
    mesh = plsc.VectorSubcoreMesh(core_axis_name="c", subcore_axis_name="s")

    @functools.partial(
        pl.kernel,
        mesh=mesh,
        out_type=jax.ShapeDtypeStruct((B,), jnp.float32),
        scratch_types=[
            pltpu.VMEM((b_per_w,), jnp.int32),
            pltpu.VMEM((b_per_w,), jnp.float32),
            pltpu.SemaphoreType.DMA,
        ],
    )
    def gather(idx_hbm, table_hbm, out_hbm, idx_v, rows_v, sem):
        wid = lax.axis_index("s") * NC + lax.axis_index("c")
        base = wid * b_per_w
        pltpu.sync_copy(idx_hbm.at[pl.ds(base, b_per_w)], idx_v)
        pltpu.async_copy(table_hbm.at[idx_v], rows_v, sem).wait()
        pltpu.sync_copy(rows_v, out_hbm.at[pl.ds(base, b_per_w)])

    return gather


def kernel(ti, tjs, ci, Cs, emb):
    B = ti.shape[0]
    V = emb.shape[0]
    tlast = tjs[:, -1:]
    emb_row = emb.reshape(1, V)
    mUT = _outer_product_t(ti.reshape(1, B), tlast.reshape(1, B), emb)
    mU = mUT.T
    mu_c = _make_sc_gather(B)(ci.reshape(B), emb.reshape(V))
    return mu_c.reshape(B, 1), mU


# speedup vs baseline: 1.0675x; 1.0432x over previous
import functools

import jax
import jax.numpy as jnp
from jax import lax
from jax.experimental import pallas as pl
from jax.experimental.pallas import tpu as pltpu
from jax.experimental.pallas import tpu_sc as plsc

BLOCK_B = 16
NBUF = 6
COL_CHUNKS = ((0, 25088), (25088, 25088), (50176, 25088), (75264, 24736))


def _outer_body(ti_ref, tl_ref, emb_ref, out_hbm, buf, sems):
    i = pl.program_id(0)
    n = pl.num_programs(0)
    slot = jax.lax.rem(i, NBUF)

    def copy(step, s, sub):
        c0, w = COL_CHUNKS[sub]
        return pltpu.make_async_copy(
            buf.at[s, :, pl.ds(c0, w)],
            out_hbm.at[pl.ds(step * BLOCK_B, BLOCK_B), pl.ds(c0, w)],
            sems.at[s, sub],
        )

    @pl.when(i >= NBUF)
    def _():
        for sub in range(len(COL_CHUNKS)):
            copy(i - NBUF, slot, sub).wait()

    rows = pl.ds(i * BLOCK_B, BLOCK_B)
    dts = ti_ref[rows, :] - tl_ref[rows, :]
    buf[slot] = dts * emb_ref[...]
    for sub in range(len(COL_CHUNKS)):
        copy(i, slot, sub).start()

    @pl.when(i == n - 1)
    def _():
        for k in range(NBUF):
            for sub in range(len(COL_CHUNKS)):
                copy(i, jax.lax.rem(i - k, NBUF), sub).wait()


def _outer_product(ti, tlast, emb_row):
    B = ti.shape[0]
    V = emb_row.shape[1]
    grid = B // BLOCK_B
    return pl.pallas_call(
        _outer_body,
        grid=(grid,),
        in_specs=[
            pl.BlockSpec((B, 1), lambda i: (0, 0)),
            pl.BlockSpec((B, 1), lambda i: (0, 0)),
            pl.BlockSpec((1, V), lambda i: (0, 0)),
        ],
        out_specs=pl.BlockSpec(memory_space=pl.ANY),
        out_shape=jax.ShapeDtypeStruct((B, V), jnp.float32),
        scratch_shapes=[
            pltpu.VMEM((NBUF, BLOCK_B, V), jnp.float32),
            pltpu.SemaphoreType.DMA((NBUF, len(COL_CHUNKS))),
        ],
    )(ti, tlast, emb_row)


BLOCK_V = 5000


def _outer_t_body(ti_ref, tl_ref, emb_ref, out_ref):
    dts = ti_ref[...] - tl_ref[...]
    out_ref[...] = emb_ref[...] * dts


def _outer_product_t(ti_row, tl_row, emb):
    V = emb.shape[0]
    B = ti_row.shape[1]
    grid = pl.cdiv(V, BLOCK_V)
    return pl.pallas_call(
        _outer_t_body,
        grid=(grid,),
        in_specs=[
            pl.BlockSpec((1, B), lambda i: (0, 0)),
            pl.BlockSpec((1, B), lambda i: (0, 0)),
            pl.BlockSpec((BLOCK_V, 1), lambda i: (i, 0)),
        ],
        out_specs=pl.BlockSpec((BLOCK_V, B), lambda i: (i, 0)),
        out_shape=jax.ShapeDtypeStruct((V, B), jnp.float32),
    )(ti_row, tl_row, emb)


@functools.lru_cache(maxsize=None)
def _make_sc_gather(B):
    info = plsc.get_sparse_core_info()
    NC, NS = info.num_cores, info.num_subcores
    NW = NC * NS
    b_per_w = B // NW
    mesh = plsc.VectorSubcoreMesh(core_axis_name="c", subcore_axis_name="s")

    @functools.partial(
        pl.kernel,
        mesh=mesh,
        out_type=jax.ShapeDtypeStruct((B,), jnp.float32),
        scratch_types=[
            pltpu.VMEM((b_per_w,), jnp.int32),
            pltpu.VMEM((b_per_w,), jnp.float32),
            pltpu.SemaphoreType.DMA,
        ],
    )
    def gather(idx_hbm, table_hbm, out_hbm, idx_v, rows_v, sem):
        wid = lax.axis_index("s") * NC + lax.axis_index("c")
        base = wid * b_per_w
        pltpu.sync_copy(idx_hbm.at[pl.ds(base, b_per_w)], idx_v)
        pltpu.async_copy(table_hbm.at[idx_v], rows_v, sem).wait()
        pltpu.sync_copy(rows_v, out_hbm.at[pl.ds(base, b_per_w)])

    return gather


def kernel(ti, tjs, ci, Cs, emb):
    B = ti.shape[0]
    V = emb.shape[0]
    tlast = tjs[:, -1:]
    emb_row = emb.reshape(1, V)
    mUT = _outer_product_t(ti.reshape(1, B), tlast.reshape(1, B), emb)
    mU = mUT.T
    mu_c = jnp.take(emb, ci, axis=0).squeeze(1)
    return mu_c, mU
